# baseline (speedup 1.0000x reference)
import functools

import jax
import jax.numpy as jnp
from jax import lax
from jax.experimental import pallas as pl
from jax.experimental.pallas import tpu as pltpu
from jax.experimental.pallas import tpu_sc as plsc

MAX_NUM_TILES = 4
HIDDEN_SIZE = 1280
NUM_PATCHES = 1025
BATCH = 16

NC = 2
NS = 16
NW = NC * NS
SEGS = BATCH * MAX_NUM_TILES
SEG_PER_W = SEGS // NW
SEG_ELEMS = NUM_PATCHES * HIDDEN_SIZE
ROWS = 25
CPS = NUM_PATCHES // ROWS
CHUNK = ROWS * HIDDEN_SIZE
CH = SEG_PER_W * CPS
NBUF = 3
LANES = 16
VPR = HIDDEN_SIZE // LANES
IDXW = 8


def _sc_body(hid_ref, idx_ref, table_ref, g_ref, out_ref,
             idx_v, g_v, rows_v, emb, bufs, in_sems, out_sems, gsem):
    wid = lax.axis_index("s") * NC + lax.axis_index("c")

    pltpu.sync_copy(idx_ref, idx_v)
    pltpu.sync_copy(g_ref, g_v)
    gv = g_v[pl.ds(0, LANES)]

    idx_w = idx_v.at[pl.ds(wid * IDXW, IDXW)]
    pltpu.make_async_copy(table_ref.at[idx_w], rows_v, gsem).start()
    pltpu.make_async_copy(table_ref.at[idx_w], rows_v, gsem).wait()
    for sj in range(SEG_PER_W):
        for j in range(VPR):
            emb[pl.ds(sj * HIDDEN_SIZE + j * LANES, LANES)] = (
                gv * rows_v[sj, pl.ds(j * LANES, LANES)])

    def hbm_off(c):
        sj = c // CPS
        l = c - sj * CPS
        s = wid * SEG_PER_W + sj
        return s * SEG_ELEMS + l * CHUNK

    def in_copy(c, slot):
        return pltpu.make_async_copy(
            hid_ref.at[pl.ds(hbm_off(c), CHUNK)],
            bufs.at[pl.ds(slot * CHUNK, CHUNK)], in_sems.at[slot])

    def out_copy(c, slot):
        return pltpu.make_async_copy(
            bufs.at[pl.ds(slot * CHUNK, CHUNK)],
            out_ref.at[pl.ds(hbm_off(c), CHUNK)], out_sems.at[slot])

    for c in range(NBUF):
        in_copy(c, c % NBUF).start()

    @pl.loop(0, CH)
    def _chunk(c):
        slot = c % NBUF

        @pl.when(c >= NBUF - 1)
        def _free_slot():
            out_copy(c - (NBUF - 1), (c + 1) % NBUF).wait()

        @pl.when(jnp.logical_and(c >= NBUF - 1, c + 1 < CH))
        def _prefetch():
            in_copy(c + 1, (c + 1) % NBUF).start()

        in_copy(c, slot).wait()
        sj = c // CPS

        out_copy(c, slot).start()

    for k in range(NBUF - 1):
        c = CH - (NBUF - 1) + k
        out_copy(c, c % NBUF).wait()


def kernel(hidden_state, aspect_ratio_ids, embedding_table, gate):
    ids = aspect_ratio_ids.astype(jnp.int32)
    g16 = jnp.full((LANES,), jnp.tanh(gate[0]), dtype=jnp.float32)
    flat = hidden_state.reshape(-1)
    table36 = embedding_table.reshape(
        embedding_table.shape[0] * MAX_NUM_TILES, HIDDEN_SIZE)
    seg_rows = (jnp.repeat(ids, MAX_NUM_TILES) * MAX_NUM_TILES
                + jnp.tile(jnp.arange(MAX_NUM_TILES, dtype=jnp.int32), BATCH))
    idx2d = jnp.zeros((NW, IDXW), dtype=jnp.int32)
    idx2d = idx2d.at[:, :SEG_PER_W].set(seg_rows.reshape(NW, SEG_PER_W))
    idx_flat = idx2d.reshape(-1)

    mesh = plsc.VectorSubcoreMesh(core_axis_name="c", subcore_axis_name="s")
    run = functools.partial(
        pl.kernel,
        out_type=jax.ShapeDtypeStruct(flat.shape, flat.dtype),
        mesh=mesh,
        scratch_types=[
            pltpu.VMEM((NW * IDXW,), jnp.int32),
            pltpu.VMEM((LANES,), jnp.float32),
            pltpu.VMEM((IDXW, HIDDEN_SIZE), jnp.float32),
            pltpu.VMEM((SEG_PER_W * HIDDEN_SIZE,), jnp.float32),
            pltpu.VMEM((NBUF * CHUNK,), jnp.float32),
            pltpu.SemaphoreType.DMA((NBUF,)),
            pltpu.SemaphoreType.DMA((NBUF,)),
            pltpu.SemaphoreType.DMA,
        ],
    )(_sc_body)
    out = run(flat, idx_flat, table36, g16)
    return out.reshape(hidden_state.shape)


# speedup vs baseline: 9.0077x; 9.0077x over previous
import jax
import jax.numpy as jnp
from jax.experimental import pallas as pl
from jax.experimental.pallas import tpu as pltpu

MAX_NUM_TILES = 4
HIDDEN_SIZE = 1280
NUM_PATCHES = 1025


def _body(ids_ref, gate_ref, table_ref, hid_ref, out_ref):
    b = pl.program_id(0)
    t = pl.program_id(1)
    h = pl.program_id(2)
    row = ids_ref[b]
    g = jnp.tanh(gate_ref[0])
    emb = table_ref[row, t, pl.ds(h * 640, 640)]
    out_ref[...] = hid_ref[...] + (g * emb)[None, None, None, :]


def kernel(hidden_state, aspect_ratio_ids, embedding_table, gate):
    batch = hidden_state.shape[0]
    ids = aspect_ratio_ids.astype(jnp.int32)
    table4d = embedding_table.reshape(
        embedding_table.shape[0], MAX_NUM_TILES, HIDDEN_SIZE)

    return pl.pallas_call(
        _body,
        grid=(batch, MAX_NUM_TILES, 2),
        in_specs=[
            pl.BlockSpec(memory_space=pltpu.SMEM),
            pl.BlockSpec(memory_space=pltpu.SMEM),
            pl.BlockSpec(memory_space=pltpu.VMEM),
            pl.BlockSpec((1, 1, NUM_PATCHES, 640),
                         lambda b, t, h: (b, t, 0, h)),
        ],
        out_specs=pl.BlockSpec((1, 1, NUM_PATCHES, 640),
                               lambda b, t, h: (b, t, 0, h)),
        out_shape=jax.ShapeDtypeStruct(hidden_state.shape, hidden_state.dtype),
        compiler_params=pltpu.CompilerParams(
            dimension_semantics=("arbitrary", "arbitrary", "arbitrary"),
            vmem_limit_bytes=63 * 1024 * 1024,
        ),
    )(ids, gate, table4d, hidden_state)


# speedup vs baseline: 9.0905x; 1.0092x over previous
import jax
import jax.numpy as jnp
from jax.experimental import pallas as pl
from jax.experimental.pallas import tpu as pltpu

MAX_NUM_TILES = 4
HIDDEN_SIZE = 1280
NUM_PATCHES = 1025


def _body(ids_ref, gate_ref, table_ref, hid_ref, out_ref):
    b = pl.program_id(0)
    th = pl.program_id(1)
    row = ids_ref[b]
    g = jnp.tanh(gate_ref[0])
    emb = table_ref[row, pl.ds(th * 2, 2)]
    out_ref[...] = hid_ref[...] + (g * emb)[None]


def kernel(hidden_state, aspect_ratio_ids, embedding_table, gate):
    batch = hidden_state.shape[0]
    ids = aspect_ratio_ids.astype(jnp.int32)
    table4d = embedding_table.reshape(
        embedding_table.shape[0], MAX_NUM_TILES, 1, HIDDEN_SIZE)

    return pl.pallas_call(
        _body,
        grid=(batch, MAX_NUM_TILES // 2),
        in_specs=[
            pl.BlockSpec(memory_space=pltpu.SMEM),
            pl.BlockSpec(memory_space=pltpu.SMEM),
            pl.BlockSpec(memory_space=pltpu.VMEM),
            pl.BlockSpec((1, 2, NUM_PATCHES, HIDDEN_SIZE),
                         lambda b, th: (b, th, 0, 0)),
        ],
        out_specs=pl.BlockSpec((1, 2, NUM_PATCHES, HIDDEN_SIZE),
                               lambda b, th: (b, th, 0, 0)),
        out_shape=jax.ShapeDtypeStruct(hidden_state.shape, hidden_state.dtype),
        compiler_params=pltpu.CompilerParams(
            dimension_semantics=("parallel", "parallel"),
            vmem_limit_bytes=63 * 1024 * 1024,
        ),
    )(ids, gate, table4d, hidden_state)
